# baseline (speedup 1.0000x reference)
import functools

import jax
import jax.numpy as jnp
from jax import lax
from jax.experimental import pallas as pl
from jax.experimental.pallas import tpu as pltpu
from jax.experimental.pallas import tpu_sc as plsc

_CHUNK = 128
_GROUP = 8


def _make_gather(V, D, N):
    info = plsc.get_sparse_core_info()
    NC, NS = info.num_cores, info.num_subcores
    NW = NC * NS
    rows_per_it = _GROUP * _CHUNK
    assert N % (NW * rows_per_it) == 0
    n_per_w = N // NW
    n_it = n_per_w // rows_per_it

    mesh = plsc.VectorSubcoreMesh(core_axis_name="c", subcore_axis_name="s")

    @functools.partial(
        pl.kernel,
        out_type=jax.ShapeDtypeStruct((N, D), jnp.float32),
        mesh=mesh,
        scratch_types=[
            pltpu.VMEM((_GROUP, _CHUNK), jnp.int32),
            pltpu.VMEM((rows_per_it, D), jnp.float32),
            pltpu.SemaphoreType.DMA,
        ],
        compiler_params=pltpu.CompilerParams(use_tc_tiling_on_sc=False),
    )
    def gather_kernel(w_hbm, idx_hbm, out_hbm, idx_buf, rows_buf, sem):
        wid = lax.axis_index("s") * NC + lax.axis_index("c")
        wbase = wid * n_per_w

        def body(i, carry):
            base = pl.multiple_of(wbase + i * rows_per_it, rows_per_it)
            irow = pl.multiple_of(base // _CHUNK, _GROUP)
            pltpu.sync_copy(idx_hbm.at[pl.ds(irow, _GROUP)], idx_buf)
            copies = []
            for j in range(_GROUP):
                copies.append(
                    pltpu.async_copy(
                        w_hbm.at[idx_buf.at[j]],
                        rows_buf.at[pl.ds(j * _CHUNK, _CHUNK)],
                        sem,
                    )
                )
            for c in copies:
                c.wait()
            pltpu.sync_copy(rows_buf, out_hbm.at[pl.ds(base, rows_per_it)])
            return carry

        lax.fori_loop(0, n_it, body, 0)

    return gather_kernel


def kernel(q, W):
    B, L = q.shape
    V, D = W.shape
    N = B * L
    idx2d = q.reshape(N // _CHUNK, _CHUNK).astype(jnp.int32)
    out = _make_gather(V, D, N)(W, idx2d)
    return out.reshape(B, L, D)


# speedup vs baseline: 1.0098x; 1.0098x over previous
import functools

import jax
import jax.numpy as jnp
from jax import lax
from jax.experimental import pallas as pl
from jax.experimental.pallas import tpu as pltpu
from jax.experimental.pallas import tpu_sc as plsc

_CHUNK = 128
_GROUP = 8


def _make_gather(V, D, N):
    info = plsc.get_sparse_core_info()
    NC, NS = info.num_cores, info.num_subcores
    NW = NC * NS
    rows_per_it = _GROUP * _CHUNK
    assert N % (NW * rows_per_it) == 0
    n_per_w = N // NW
    n_it = n_per_w // rows_per_it

    mesh = plsc.VectorSubcoreMesh(core_axis_name="c", subcore_axis_name="s")

    half = rows_per_it // 2
    hg = _GROUP // 2

    @functools.partial(
        pl.kernel,
        out_type=jax.ShapeDtypeStruct((N, D), jnp.float32),
        mesh=mesh,
        scratch_types=[
            pltpu.VMEM((_GROUP, _CHUNK), jnp.int32),
            pltpu.VMEM((half, D), jnp.float32),
            pltpu.VMEM((half, D), jnp.float32),
            pltpu.SemaphoreType.DMA,
            pltpu.SemaphoreType.DMA,
        ],
        compiler_params=pltpu.CompilerParams(use_tc_tiling_on_sc=False),
    )
    def gather_kernel(w_hbm, idx_hbm, out_hbm, idx_buf, rows0, rows1, gsem, wsem):
        wid = lax.axis_index("s") * NC + lax.axis_index("c")
        wbase = wid * n_per_w
        bufs = (rows0, rows1)

        def body(i, carry):
            base = pl.multiple_of(wbase + i * rows_per_it, rows_per_it)
            irow = pl.multiple_of(base // _CHUNK, _GROUP)
            pltpu.sync_copy(idx_hbm.at[pl.ds(irow, _GROUP)], idx_buf)
            for s in range(2):
                buf = bufs[s]
                @pl.when(i > 0)
                def _():
                    pltpu.make_async_copy(
                        buf, out_hbm.at[pl.ds(0, half)], wsem
                    ).wait()
                copies = [
                    pltpu.async_copy(
                        w_hbm.at[idx_buf.at[s * hg + j]],
                        buf.at[pl.ds(j * _CHUNK, _CHUNK)],
                        gsem,
                    )
                    for j in range(hg)
                ]
                for c in copies:
                    c.wait()
                pltpu.async_copy(
                    buf, out_hbm.at[pl.ds(base + s * half, half)], wsem
                )
            return carry

        lax.fori_loop(0, n_it, body, 0)
        for s in range(2):
            pltpu.make_async_copy(bufs[s], out_hbm.at[pl.ds(0, half)], wsem).wait()

    return gather_kernel


def kernel(q, W):
    B, L = q.shape
    V, D = W.shape
    N = B * L
    idx2d = q.reshape(N // _CHUNK, _CHUNK).astype(jnp.int32)
    out = _make_gather(V, D, N)(W, idx2d)
    return out.reshape(B, L, D)
